# baseline (speedup 1.0000x reference)
import functools

import jax
import jax.numpy as jnp
from jax import lax
from jax.experimental import pallas as pl
from jax.experimental.pallas import tpu as pltpu
from jax.experimental.pallas import tpu_sc as plsc

EMBED_DIM = 2048
NUM_CORES = 2
NUM_SUBCORES = 16
NUM_TILES = NUM_CORES * NUM_SUBCORES
ROWS_PER_CHUNK = 16


@functools.partial(jax.jit, static_argnames=("num_chunks",))
def _sc_embed(embed_weight, idx, num_chunks):
    rows_per_tile = num_chunks * ROWS_PER_CHUNK
    total_rows = NUM_TILES * rows_per_tile
    mesh = plsc.VectorSubcoreMesh(core_axis_name="c", subcore_axis_name="s")

    @functools.partial(
        pl.kernel,
        out_type=jax.ShapeDtypeStruct((total_rows, EMBED_DIM), jnp.float32),
        mesh=mesh,
        scratch_types=[
            pltpu.VMEM((num_chunks, ROWS_PER_CHUNK), jnp.int32),
            pltpu.VMEM((ROWS_PER_CHUNK, EMBED_DIM), jnp.float32),
            pltpu.VMEM((ROWS_PER_CHUNK, EMBED_DIM), jnp.float32),
            pltpu.SemaphoreType.DMA,
            pltpu.SemaphoreType.DMA,
            pltpu.SemaphoreType.DMA,
            pltpu.SemaphoreType.DMA,
        ],
    )
    def k(table_hbm, idx_hbm, out_hbm, idx_v, buf0, buf1, gs0, gs1, ss0, ss1):
        wid = lax.axis_index("s") * NUM_CORES + lax.axis_index("c")
        pltpu.sync_copy(idx_hbm.at[wid], idx_v)
        base = wid * rows_per_tile
        R = ROWS_PER_CHUNK

        def fire_gather(j, buf, sem):
            pltpu.async_copy(table_hbm.at[idx_v.at[j]], buf, sem)

        def wait_gather(j, buf, sem):
            pltpu.make_async_copy(table_hbm.at[idx_v.at[j]], buf, sem).wait()

        def fire_store(j, buf, sem):
            pltpu.async_copy(buf, out_hbm.at[pl.ds(base + j * R, R)], sem)

        def wait_store(j, buf, sem):
            pltpu.make_async_copy(
                buf, out_hbm.at[pl.ds(base + j * R, R)], sem
            ).wait()

        fire_gather(0, buf0, gs0)
        fire_gather(1, buf1, gs1)

        @pl.loop(0, num_chunks - 2, step=2)
        def _(j):
            wait_gather(j, buf0, gs0)
            fire_store(j, buf0, ss0)
            wait_gather(j + 1, buf1, gs1)
            fire_store(j + 1, buf1, ss1)
            wait_store(j, buf0, ss0)
            fire_gather(j + 2, buf0, gs0)
            wait_store(j + 1, buf1, ss1)
            fire_gather(j + 3, buf1, gs1)

        jl = num_chunks - 2
        wait_gather(jl, buf0, gs0)
        fire_store(jl, buf0, ss0)
        wait_gather(jl + 1, buf1, gs1)
        fire_store(jl + 1, buf1, ss1)
        wait_store(jl, buf0, ss0)
        wait_store(jl + 1, buf1, ss1)

    return k(embed_weight, idx)


def kernel(input_ids, embed_weight):
    batch, seq = input_ids.shape
    total = batch * seq
    num_chunks = total // (NUM_TILES * ROWS_PER_CHUNK)
    idx = input_ids.reshape(NUM_TILES, num_chunks, ROWS_PER_CHUNK)
    out = _sc_embed(embed_weight, idx, num_chunks)
    return out.reshape(batch, seq, embed_weight.shape[1])


# speedup vs baseline: 1.0670x; 1.0670x over previous
import functools

import jax
import jax.numpy as jnp
from jax import lax
from jax.experimental import pallas as pl
from jax.experimental.pallas import tpu as pltpu
from jax.experimental.pallas import tpu_sc as plsc

NUM_CORES = 2
NUM_SUBCORES = 16
NUM_TILES = NUM_CORES * NUM_SUBCORES
ROWS_PER_CHUNK = 16


@functools.partial(jax.jit, static_argnames=("batch", "seq", "dim"))
def _sc_embed(embed_weight, input_ids, batch, seq, dim):
    rows_per_tile = (batch * seq) // NUM_TILES
    num_chunks = rows_per_tile // ROWS_PER_CHUNK
    tiles_per_batch_row = seq // rows_per_tile
    mesh = plsc.VectorSubcoreMesh(core_axis_name="c", subcore_axis_name="s")

    @functools.partial(
        pl.kernel,
        out_type=jax.ShapeDtypeStruct((batch, seq, dim), jnp.float32),
        mesh=mesh,
        scratch_types=[
            pltpu.VMEM((rows_per_tile,), jnp.int32),
            pltpu.VMEM((ROWS_PER_CHUNK, dim), jnp.float32),
            pltpu.VMEM((ROWS_PER_CHUNK, dim), jnp.float32),
            pltpu.SemaphoreType.DMA,
            pltpu.SemaphoreType.DMA,
        ],
    )
    def k(table_hbm, idx_hbm, out_hbm, idx_v, buf0, buf1, gs0, gs1):
        wid = lax.axis_index("s") * NUM_CORES + lax.axis_index("c")
        b = wid // tiles_per_batch_row
        off = (wid % tiles_per_batch_row) * rows_per_tile
        pltpu.sync_copy(idx_hbm.at[b, pl.ds(off, rows_per_tile)], idx_v)
        R = ROWS_PER_CHUNK

        def fire_gather(j, buf, sem):
            pltpu.async_copy(
                table_hbm.at[idx_v.at[pl.ds(j * R, R)]], buf, sem
            )

        def wait_gather(j, buf, sem):
            pltpu.make_async_copy(
                table_hbm.at[idx_v.at[pl.ds(j * R, R)]], buf, sem
            ).wait()

        def sync_store(j, buf):
            pltpu.sync_copy(buf, out_hbm.at[b, pl.ds(off + j * R, R)])

        fire_gather(0, buf0, gs0)

        @pl.loop(0, num_chunks - 2, step=2)
        def _(j):
            fire_gather(j + 1, buf1, gs1)
            wait_gather(j, buf0, gs0)
            sync_store(j, buf0)
            fire_gather(j + 2, buf0, gs0)
            wait_gather(j + 1, buf1, gs1)
            sync_store(j + 1, buf1)

        jl = num_chunks - 2
        fire_gather(jl + 1, buf1, gs1)
        wait_gather(jl, buf0, gs0)
        sync_store(jl, buf0)
        wait_gather(jl + 1, buf1, gs1)
        sync_store(jl + 1, buf1)

    return k(embed_weight, input_ids)


def kernel(input_ids, embed_weight):
    batch, seq = input_ids.shape
    dim = embed_weight.shape[1]
    return _sc_embed(embed_weight, input_ids, batch, seq, dim)
